# Initial kernel scaffold; baseline (speedup 1.0000x reference)
import jax
import jax.numpy as jnp
from jax.experimental import pallas as pl


def kernel(input, table):
    raise NotImplementedError("write your pallas kernel here")


# speedup vs baseline: 2.9316x; 2.9316x over previous
import functools

import jax
import jax.numpy as jnp
from jax import lax
from jax.experimental import pallas as pl
from jax.experimental.pallas import tpu as pltpu
from jax.experimental.pallas import tpu_sc as plsc

NUM_EMB = 1_000_000
DIM = 32
HIST = 50
BATCH = 16384

NC = 2
NS = 16
NW = NC * NS

BAGS_PER_W = BATCH // NW
CHUNK = 32
NCHUNK = BAGS_PER_W // CHUNK
ROWS = CHUNK * HIST
GROUP = 100
GPC = ROWS // GROUP
LANES = 16
SCALE = 1.0 / HIST

_mesh = plsc.VectorSubcoreMesh(
    core_axis_name="c", subcore_axis_name="s", num_cores=NC, num_subcores=NS
)


@functools.partial(
    pl.kernel,
    out_type=jax.ShapeDtypeStruct((BATCH, DIM), jnp.float32),
    mesh=_mesh,
    scratch_types=[
        pltpu.VMEM((2, GPC, GROUP), jnp.int32),
        pltpu.VMEM((2, ROWS, DIM), jnp.float32),
        pltpu.VMEM((CHUNK, DIM), jnp.float32),
        pltpu.SemaphoreType.DMA,
        pltpu.SemaphoreType.DMA,
    ],
    compiler_params=pltpu.CompilerParams(use_tc_tiling_on_sc=False),
)
def _embbag(idx_hbm, table_hbm, out_hbm, idx_v, rows_v, out_v, sem0, sem1):
    cid = lax.axis_index("c")
    sid = lax.axis_index("s")
    wid = sid * NC + cid
    gbase = wid * (BAGS_PER_W * HIST // GROUP)
    bagbase = wid * BAGS_PER_W
    sems = (sem0, sem1)

    def stage(c, buf):
        pltpu.sync_copy(idx_hbm.at[pl.ds(gbase + c * GPC, GPC)], idx_v.at[buf])
        for g in range(GPC):
            pltpu.async_copy(
                table_hbm.at[idx_v.at[buf, g]],
                rows_v.at[buf, pl.ds(g * GROUP, GROUP)],
                sems[buf],
            )

    def drain(buf):
        pltpu.make_async_copy(
            table_hbm.at[pl.ds(0, ROWS)], rows_v.at[buf], sems[buf]
        ).wait()

    def compute(c, buf):
        def bag_body(b, carry):
            r0 = b * HIST
            acc0 = rows_v[buf, r0, pl.ds(0, LANES)]
            acc1 = rows_v[buf, r0, pl.ds(LANES, LANES)]
            for j in range(1, HIST):
                acc0 = acc0 + rows_v[buf, r0 + j, pl.ds(0, LANES)]
                acc1 = acc1 + rows_v[buf, r0 + j, pl.ds(LANES, LANES)]
            out_v[b, pl.ds(0, LANES)] = acc0 * SCALE
            out_v[b, pl.ds(LANES, LANES)] = acc1 * SCALE
            return carry
        lax.fori_loop(0, CHUNK, bag_body, 0)
        pltpu.sync_copy(out_v, out_hbm.at[pl.ds(bagbase + c * CHUNK, CHUNK)])

    stage(0, 0)

    @pl.loop(0, NCHUNK, step=2)
    def _chunk_pair(c):
        for buf in range(2):
            cc = c + buf

            @pl.when(cc + 1 < NCHUNK)
            def _():
                stage(cc + 1, 1 - buf)

            drain(buf)
            compute(cc, buf)


def kernel(input, table):
    idx = input.astype(jnp.int32).reshape(BATCH * HIST // GROUP, GROUP)
    return _embbag(idx, table)
